# Initial kernel scaffold; baseline (speedup 1.0000x reference)
import jax
import jax.numpy as jnp
from jax.experimental import pallas as pl


def kernel(x, Wg, bg, W1, b1, W2, b2):
    raise NotImplementedError("write your pallas kernel here")


# speedup vs baseline: 1.7154x; 1.7154x over previous
import functools

import jax
import jax.numpy as jnp
from jax.experimental import pallas as pl
from jax.experimental.pallas import tpu as pltpu

E = 8
D = 768
H = 768
O = 768
T = 4096
TBLK = 256
NT = T // TBLK


def _moe_dense_body(x_ref, wg_ref, bg_ref, w1_ref, b1_ref, w2_ref, b2_ref,
                    out_ref):
    xb = x_ref[...]
    logits = jnp.dot(xb.astype(jnp.bfloat16),
                     wg_ref[...].astype(jnp.bfloat16),
                     preferred_element_type=jnp.float32) + bg_ref[...]
    m = jnp.max(logits, axis=-1, keepdims=True)
    gate = 1.0 / jnp.sum(jnp.exp(logits - m), axis=-1, keepdims=True)
    eid = jnp.argmax(logits, axis=-1)

    xbf = xb.astype(jnp.bfloat16)
    out_ref[...] = jnp.zeros_like(out_ref)
    for e in range(E):
        h = jnp.dot(xbf, w1_ref[e], preferred_element_type=jnp.float32)
        h = jnp.maximum(h + b1_ref[e], 0.0)
        o = jnp.dot(h.astype(jnp.bfloat16), w2_ref[e],
                    preferred_element_type=jnp.float32) + b2_ref[e]
        w = jnp.where(eid[:, None] == e, gate, 0.0)
        out_ref[...] += w * o


@functools.partial(jax.jit, static_argnums=())
def kernel(x, Wg, bg, W1, b1, W2, b2):
    w1_bf = W1.astype(jnp.bfloat16)
    w2_bf = W2.astype(jnp.bfloat16)
    bg2 = bg.reshape(1, E)
    return pl.pallas_call(
        _moe_dense_body,
        grid=(NT,),
        in_specs=[
            pl.BlockSpec((TBLK, D), lambda t: (t, 0)),
            pl.BlockSpec((D, E), lambda t: (0, 0)),
            pl.BlockSpec((1, E), lambda t: (0, 0)),
            pl.BlockSpec((E, D, H), lambda t: (0, 0, 0)),
            pl.BlockSpec((E, H), lambda t: (0, 0)),
            pl.BlockSpec((E, H, O), lambda t: (0, 0, 0)),
            pl.BlockSpec((E, O), lambda t: (0, 0)),
        ],
        out_specs=pl.BlockSpec((TBLK, O), lambda t: (t, 0)),
        out_shape=jax.ShapeDtypeStruct((T, O), jnp.float32),
    )(x, Wg, bg2, w1_bf, b1, w2_bf, b2)


# speedup vs baseline: 1.8142x; 1.0576x over previous
import functools

import jax
import jax.numpy as jnp
from jax import lax
from jax.experimental import pallas as pl
from jax.experimental.pallas import tpu as pltpu
from jax.experimental.pallas import tpu_sc as plsc

E = 8
D = 768
H = 768
O = 768
T = 4096

CH = 512
NC = T // CH
TBLK = 256
TP = T + E * TBLK
NB = TP // TBLK

SC_CORES = 2
SC_SUBCORES = 16
NW = SC_CORES * SC_SUBCORES
TOK_W = T // NW
GW = 128


def _gating(x, wg, bg):
    logits = jnp.dot(x.astype(jnp.bfloat16), wg.astype(jnp.bfloat16),
                     preferred_element_type=jnp.float32) + bg
    m = jnp.max(logits, axis=-1, keepdims=True)
    gate = 1.0 / jnp.sum(jnp.exp(logits - m), axis=-1, keepdims=True)
    eid = jnp.argmax(logits, axis=-1)
    return gate, eid


def _route_body(x_ref, wg_ref, bg_ref, slot_ref, gate8_ref, bexp_ref,
                cnt_ref, eid_ref, grank_ref):
    i = pl.program_id(0)

    @pl.when(i == 0)
    def _init():
        cnt_ref[...] = jnp.zeros_like(cnt_ref)

    @pl.when(i < NC)
    def _phase0():
        gate, eid = _gating(x_ref[...], wg_ref[...], bg_ref[...])
        onehot = (eid[:, None] ==
                  lax.broadcasted_iota(jnp.int32, (CH, E), 1)
                  ).astype(jnp.float32)
        r = lax.broadcasted_iota(jnp.int32, (CH, CH), 0)
        c = lax.broadcasted_iota(jnp.int32, (CH, CH), 1)
        tri = (c < r).astype(jnp.bfloat16)
        rank = jnp.dot(tri, onehot.astype(jnp.bfloat16),
                       preferred_element_type=jnp.float32)
        base = cnt_ref[...]
        grank = jnp.sum((rank + base) * onehot, axis=1)
        eid_ref[pl.ds(i, 1), :] = eid[None, :]
        grank_ref[pl.ds(i, 1), :] = grank.astype(jnp.int32)[None, :]
        gate8_ref[...] = jnp.broadcast_to(gate, (CH, GW))
        cnt_ref[...] = base + jnp.sum(onehot, axis=0, keepdims=True)

    @pl.when(i >= NC)
    def _phase1():
        j = i - NC
        counts = cnt_ref[...]
        padded = jnp.floor((counts + (TBLK - 1)) / TBLK) * TBLK
        tr = lax.broadcasted_iota(jnp.int32, (E, E), 0)
        tc = lax.broadcasted_iota(jnp.int32, (E, E), 1)
        tri_incl = (tr <= tc).astype(jnp.bfloat16)
        ends = jnp.dot(padded.astype(jnp.bfloat16), tri_incl,
                       preferred_element_type=jnp.float32)
        pad_off = ends - padded
        eid = eid_ref[pl.ds(j, 1), :].reshape(CH)
        onehot = (eid[:, None] ==
                  lax.broadcasted_iota(jnp.int32, (CH, E), 1)
                  ).astype(jnp.float32)
        base_slot = jnp.sum(onehot * pad_off, axis=1)
        grank = grank_ref[pl.ds(j, 1), :].reshape(CH)
        slot = base_slot.astype(jnp.int32) + grank
        slot_ref[...] = slot.reshape(1, 1, CH)
        bvals = (lax.broadcasted_iota(jnp.int32, (1, NB), 1)
                 * TBLK).astype(jnp.float32)
        acc = jnp.zeros((1, NB), jnp.int32)
        for e in range(E):
            acc += (bvals >= ends[:, e:e + 1]).astype(jnp.int32)
        bexp_ref[...] = jnp.minimum(acc, E - 1)


def _route(x, wg, bg2):
    return pl.pallas_call(
        _route_body,
        grid=(2 * NC,),
        in_specs=[
            pl.BlockSpec((CH, D), lambda i: (jnp.minimum(i, NC - 1), 0)),
            pl.BlockSpec((D, E), lambda i: (0, 0)),
            pl.BlockSpec((1, E), lambda i: (0, 0)),
        ],
        out_specs=[
            pl.BlockSpec((1, 1, CH), lambda i: (jnp.maximum(i - NC, 0), 0, 0)),
            pl.BlockSpec((CH, GW), lambda i: (jnp.minimum(i, NC - 1), 0)),
            pl.BlockSpec((1, NB), lambda i: (0, 0)),
        ],
        out_shape=[
            jax.ShapeDtypeStruct((NC, 1, CH), jnp.int32),
            jax.ShapeDtypeStruct((T, GW), jnp.float32),
            jax.ShapeDtypeStruct((1, NB), jnp.int32),
        ],
        scratch_shapes=[
            pltpu.VMEM((1, E), jnp.float32),
            pltpu.VMEM((NC, CH), jnp.int32),
            pltpu.VMEM((NC, CH), jnp.int32),
        ],
    )(x, wg, bg2)


def _mlp_body(bexp_ref, xs_ref, w1_ref, b1_ref, w2_ref, b2_ref, gs8_ref,
              ys_ref):
    del bexp_ref
    h = jnp.dot(xs_ref[...], w1_ref[0],
                preferred_element_type=jnp.float32) + b1_ref[0]
    h = jnp.maximum(h, 0.0)
    o = jnp.dot(h, w2_ref[0],
                preferred_element_type=jnp.float32) + b2_ref[0]
    ys_ref[...] = gs8_ref[:, :1] * o


def _mlp(bexp, xs, W1, b1, W2, b2, gs8):
    grid_spec = pltpu.PrefetchScalarGridSpec(
        num_scalar_prefetch=1,
        grid=(NB,),
        in_specs=[
            pl.BlockSpec((TBLK, D), lambda b, be: (b, 0)),
            pl.BlockSpec((1, D, H), lambda b, be: (be[b], 0, 0)),
            pl.BlockSpec((1, 1, H), lambda b, be: (be[b], 0, 0)),
            pl.BlockSpec((1, H, O), lambda b, be: (be[b], 0, 0)),
            pl.BlockSpec((1, 1, O), lambda b, be: (be[b], 0, 0)),
            pl.BlockSpec((TBLK, GW), lambda b, be: (b, 0)),
        ],
        out_specs=pl.BlockSpec((TBLK, O), lambda b, be: (b, 0)),
    )
    return pl.pallas_call(
        _mlp_body,
        grid_spec=grid_spec,
        out_shape=jax.ShapeDtypeStruct((TP, O), jnp.float32),
    )(bexp, xs, W1, b1.reshape(E, 1, H), W2, b2.reshape(E, 1, O), gs8)


@functools.cache
def _sc_kernels():
    mesh = plsc.VectorSubcoreMesh(
        core_axis_name="c", subcore_axis_name="s",
        num_cores=SC_CORES, num_subcores=SC_SUBCORES)

    @functools.partial(
        pl.kernel,
        out_type=(jax.ShapeDtypeStruct((TP, D), jnp.float32),
                  jax.ShapeDtypeStruct((TP, GW), jnp.float32)),
        mesh=mesh,
        scratch_types=[
            pltpu.VMEM((TOK_W,), jnp.int32),
            pltpu.VMEM((TOK_W, D), jnp.float32),
            pltpu.VMEM((TOK_W, GW), jnp.float32),
            pltpu.SemaphoreType.DMA,
            pltpu.SemaphoreType.DMA,
        ],
    )
    def sc_scatter(x_hbm, slot_hbm, gate8_hbm, xs_hbm, gs8_hbm,
                   slot_v, x_v, g8_v, sem_x, sem_g):
        wid = lax.axis_index("s") * SC_CORES + lax.axis_index("c")
        base = wid * TOK_W
        pltpu.sync_copy(slot_hbm.at[pl.ds(base, TOK_W)], slot_v)
        pltpu.sync_copy(x_hbm.at[pl.ds(base, TOK_W)], x_v)
        pltpu.sync_copy(gate8_hbm.at[pl.ds(base, TOK_W)], g8_v)
        cp_x = pltpu.async_copy(x_v, xs_hbm.at[slot_v], sem_x)
        cp_g = pltpu.async_copy(g8_v, gs8_hbm.at[slot_v], sem_g)
        cp_x.wait()
        cp_g.wait()

    @functools.partial(
        pl.kernel,
        out_type=jax.ShapeDtypeStruct((T, O), jnp.float32),
        mesh=mesh,
        scratch_types=[
            pltpu.VMEM((TOK_W,), jnp.int32),
            pltpu.VMEM((TOK_W, O), jnp.float32),
            pltpu.SemaphoreType.DMA,
        ],
    )
    def sc_gather(ys_hbm, slot_hbm, y_hbm, slot_v, y_v, sem):
        wid = lax.axis_index("s") * SC_CORES + lax.axis_index("c")
        base = wid * TOK_W
        pltpu.sync_copy(slot_hbm.at[pl.ds(base, TOK_W)], slot_v)
        pltpu.async_copy(ys_hbm.at[slot_v], y_v, sem).wait()
        pltpu.sync_copy(y_v, y_hbm.at[pl.ds(base, TOK_W)])

    return sc_scatter, sc_gather


def kernel(x, Wg, bg, W1, b1, W2, b2):
    sc_scatter, sc_gather = _sc_kernels()
    slot3, gate8, bexp2 = _route(x, Wg, bg.reshape(1, E))
    slot = slot3.reshape(T)
    bexp = bexp2.reshape(NB)
    xs, gs8 = sc_scatter(x, slot, gate8)
    ys = _mlp(bexp, xs, W1, b1, W2, b2, gs8)
    return sc_gather(ys, slot)


# speedup vs baseline: 1.9699x; 1.0858x over previous
import functools

import jax
import jax.numpy as jnp
from jax import lax
from jax.experimental import pallas as pl
from jax.experimental.pallas import tpu as pltpu
from jax.experimental.pallas import tpu_sc as plsc

E = 8
D = 768
H = 768
O = 768
T = 4096

CH = 512
NC = T // CH
TBLK = 256
TP = T + E * TBLK
NB = TP // TBLK

SC_CORES = 2
SC_SUBCORES = 16
NW = SC_CORES * SC_SUBCORES
TOK_W = T // NW
GW = 128


def _route_body(x_ref, wg_ref, bg_ref, slot_ref, gate8_ref, bexp_ref,
                cnt_ref, eid_ref, grank_ref):
    i = pl.program_id(0)

    @pl.when(i == 0)
    def _init():
        cnt_ref[...] = jnp.zeros_like(cnt_ref)

    @pl.when(i < NC)
    def _phase0():
        xb = x_ref[...]
        logits = jnp.dot(xb.astype(jnp.bfloat16),
                         wg_ref[...].astype(jnp.bfloat16),
                         preferred_element_type=jnp.float32) + bg_ref[...]
        lt = logits.T
        mT = jnp.max(lt, axis=0, keepdims=True)
        gateT = 1.0 / jnp.sum(jnp.exp(lt - mT), axis=0, keepdims=True)
        sub = lax.broadcasted_iota(jnp.int32, (E, CH), 0)
        eidT = jnp.min(jnp.where(lt == mT, sub, E), axis=0,
                       keepdims=True)
        mask = sub == eidT
        r = lax.broadcasted_iota(jnp.int32, (CH, CH), 0)
        c = lax.broadcasted_iota(jnp.int32, (CH, CH), 1)
        triu = (r < c).astype(jnp.bfloat16)
        rankT = jnp.dot(mask.astype(jnp.bfloat16), triu,
                        preferred_element_type=jnp.float32)
        base = cnt_ref[...]
        grankT = jnp.sum(jnp.where(mask, rankT + base, 0.0),
                         axis=0, keepdims=True)
        eid_ref[pl.ds(i, 1), :] = eidT
        grank_ref[pl.ds(i, 1), :] = grankT.astype(jnp.int32)
        gate8_ref[...] = jnp.broadcast_to(gateT.T, (CH, GW))
        cnt_ref[...] = base + jnp.sum(mask.astype(jnp.float32),
                                      axis=1, keepdims=True)

    @pl.when(i >= NC)
    def _phase1():
        j = i - NC
        counts = cnt_ref[...]
        padded = jnp.floor((counts + (TBLK - 1)) / TBLK) * TBLK
        tr = lax.broadcasted_iota(jnp.int32, (E, E), 0)
        tc = lax.broadcasted_iota(jnp.int32, (E, E), 1)
        tril = (tc <= tr).astype(jnp.bfloat16)
        ends = jnp.dot(tril, padded.astype(jnp.bfloat16),
                       preferred_element_type=jnp.float32)
        pad_off = ends - padded
        eidT = eid_ref[pl.ds(j, 1), :]
        sub = lax.broadcasted_iota(jnp.int32, (E, CH), 0)
        mask = sub == eidT
        base_slot = jnp.sum(jnp.where(mask, pad_off, 0.0),
                            axis=0, keepdims=True)
        slot = base_slot.astype(jnp.int32) + grank_ref[pl.ds(j, 1), :]
        slot_ref[...] = slot.reshape(1, 1, CH)
        bvals = (lax.broadcasted_iota(jnp.int32, (E, NB), 1)
                 * TBLK).astype(jnp.float32)
        acc = jnp.sum((bvals >= ends).astype(jnp.int32),
                      axis=0, keepdims=True)
        bexp_ref[...] = jnp.minimum(acc, E - 1)


def _route(x, wg, bg2):
    return pl.pallas_call(
        _route_body,
        grid=(2 * NC,),
        in_specs=[
            pl.BlockSpec((CH, D), lambda i: (jnp.minimum(i, NC - 1), 0)),
            pl.BlockSpec((D, E), lambda i: (0, 0)),
            pl.BlockSpec((1, E), lambda i: (0, 0)),
        ],
        out_specs=[
            pl.BlockSpec((1, 1, CH), lambda i: (jnp.maximum(i - NC, 0), 0, 0)),
            pl.BlockSpec((CH, GW), lambda i: (jnp.minimum(i, NC - 1), 0)),
            pl.BlockSpec((1, NB), lambda i: (0, 0)),
        ],
        out_shape=[
            jax.ShapeDtypeStruct((NC, 1, CH), jnp.int32),
            jax.ShapeDtypeStruct((T, GW), jnp.float32),
            jax.ShapeDtypeStruct((1, NB), jnp.int32),
        ],
        scratch_shapes=[
            pltpu.VMEM((E, 1), jnp.float32),
            pltpu.VMEM((NC, CH), jnp.int32),
            pltpu.VMEM((NC, CH), jnp.int32),
        ],
    )(x, wg, bg2)


def _mlp_body(bexp_ref, xs_ref, w1_ref, b1_ref, w2_ref, b2_ref, gs8_ref,
              ys_ref):
    del bexp_ref
    h = jnp.dot(xs_ref[...], w1_ref[0],
                preferred_element_type=jnp.float32) + b1_ref[0]
    h = jnp.maximum(h, 0.0)
    o = jnp.dot(h, w2_ref[0],
                preferred_element_type=jnp.float32) + b2_ref[0]
    ys_ref[...] = gs8_ref[:, :1] * o


def _mlp(bexp, xs, W1, b1, W2, b2, gs8):
    grid_spec = pltpu.PrefetchScalarGridSpec(
        num_scalar_prefetch=1,
        grid=(NB,),
        in_specs=[
            pl.BlockSpec((TBLK, D), lambda b, be: (b, 0)),
            pl.BlockSpec((1, D, H), lambda b, be: (be[b], 0, 0)),
            pl.BlockSpec((1, 1, H), lambda b, be: (be[b], 0, 0)),
            pl.BlockSpec((1, H, O), lambda b, be: (be[b], 0, 0)),
            pl.BlockSpec((1, 1, O), lambda b, be: (be[b], 0, 0)),
            pl.BlockSpec((TBLK, GW), lambda b, be: (b, 0)),
        ],
        out_specs=pl.BlockSpec((TBLK, O), lambda b, be: (b, 0)),
    )
    return pl.pallas_call(
        _mlp_body,
        grid_spec=grid_spec,
        out_shape=jax.ShapeDtypeStruct((TP, O), jnp.float32),
    )(bexp, xs, W1, b1.reshape(E, 1, H), W2, b2.reshape(E, 1, O), gs8)


@functools.cache
def _sc_kernels():
    mesh = plsc.VectorSubcoreMesh(
        core_axis_name="c", subcore_axis_name="s",
        num_cores=SC_CORES, num_subcores=SC_SUBCORES)

    @functools.partial(
        pl.kernel,
        out_type=(jax.ShapeDtypeStruct((TP, D), jnp.float32),
                  jax.ShapeDtypeStruct((TP, GW), jnp.float32)),
        mesh=mesh,
        scratch_types=[
            pltpu.VMEM((TOK_W,), jnp.int32),
            pltpu.VMEM((TOK_W, D), jnp.float32),
            pltpu.VMEM((TOK_W, GW), jnp.float32),
            pltpu.SemaphoreType.DMA,
            pltpu.SemaphoreType.DMA,
        ],
    )
    def sc_scatter(x_hbm, slot_hbm, gate8_hbm, xs_hbm, gs8_hbm,
                   slot_v, x_v, g8_v, sem_x, sem_g):
        wid = lax.axis_index("s") * SC_CORES + lax.axis_index("c")
        base = wid * TOK_W
        pltpu.sync_copy(slot_hbm.at[pl.ds(base, TOK_W)], slot_v)
        pltpu.sync_copy(x_hbm.at[pl.ds(base, TOK_W)], x_v)
        pltpu.sync_copy(gate8_hbm.at[pl.ds(base, TOK_W)], g8_v)
        cp_x = pltpu.async_copy(x_v, xs_hbm.at[slot_v], sem_x)
        cp_g = pltpu.async_copy(g8_v, gs8_hbm.at[slot_v], sem_g)
        cp_x.wait()
        cp_g.wait()

    @functools.partial(
        pl.kernel,
        out_type=jax.ShapeDtypeStruct((T, O), jnp.float32),
        mesh=mesh,
        scratch_types=[
            pltpu.VMEM((TOK_W,), jnp.int32),
            pltpu.VMEM((TOK_W, O), jnp.float32),
            pltpu.SemaphoreType.DMA,
        ],
    )
    def sc_gather(ys_hbm, slot_hbm, y_hbm, slot_v, y_v, sem):
        wid = lax.axis_index("s") * SC_CORES + lax.axis_index("c")
        base = wid * TOK_W
        pltpu.sync_copy(slot_hbm.at[pl.ds(base, TOK_W)], slot_v)
        pltpu.async_copy(ys_hbm.at[slot_v], y_v, sem).wait()
        pltpu.sync_copy(y_v, y_hbm.at[pl.ds(base, TOK_W)])

    return sc_scatter, sc_gather


def kernel(x, Wg, bg, W1, b1, W2, b2):
    sc_scatter, sc_gather = _sc_kernels()
    slot3, gate8, bexp2 = _route(x, Wg, bg.reshape(1, E))
    slot = slot3.reshape(T)
    bexp = bexp2.reshape(NB)
    xs, gs8 = sc_scatter(x, slot, gate8)
    ys = _mlp(bexp, xs, W1, b1, W2, b2, gs8)
    return sc_gather(ys, slot)
